# Initial kernel scaffold; baseline (speedup 1.0000x reference)
import jax
import jax.numpy as jnp
from jax.experimental import pallas as pl


def kernel(x, conv_w):
    raise NotImplementedError("write your pallas kernel here")


# speedup vs baseline: 1.0769x; 1.0769x over previous
import functools

import jax
import jax.numpy as jnp
from jax.experimental import pallas as pl
from jax.experimental.pallas import tpu as pltpu


def _fused_kernel(x_ref, w_ref, o_ref, *, inv_s, c_per_b):
    xb = x_ref[...].astype(jnp.float32)

    m = jnp.sum(xb, axis=1, keepdims=True) * inv_s

    rows = jax.lax.broadcasted_iota(jnp.int32, m.shape, 0)
    ch = jax.lax.rem(rows, c_per_b)
    zero = jnp.zeros((1, 1), jnp.float32)
    m_prev = jnp.where(ch == 0, 0.0,
                       jnp.concatenate([zero, m[:-1, :]], axis=0))
    m_next = jnp.where(ch == c_per_b - 1, 0.0,
                       jnp.concatenate([m[1:, :], zero], axis=0))
    z = w_ref[0] * m_prev + w_ref[1] * m + w_ref[2] * m_next
    scale = 1.0 / (1.0 + jnp.exp(-z))

    t = xb * scale
    sig = 1.0 / (1.0 + jnp.exp(-t))
    o_ref[...] = (xb * sig).astype(o_ref.dtype)


def kernel(x, conv_w):
    B, C, D, H, W = x.shape
    S = D * H * W
    R = B * C
    dtype = x.dtype

    rt = C
    for cand in (256, 128):
        if cand % C == 0 and R % cand == 0 and R // cand >= 2:
            rt = cand
            break
    n_r = R // rt

    x2 = x.reshape(R, S)
    w = conv_w.astype(jnp.float32)

    out2 = pl.pallas_call(
        functools.partial(_fused_kernel, inv_s=1.0 / float(S), c_per_b=C),
        out_shape=jax.ShapeDtypeStruct((R, S), dtype),
        grid=(n_r,),
        in_specs=[
            pl.BlockSpec((rt, S), lambda r: (r, 0)),
            pl.BlockSpec(memory_space=pltpu.SMEM),
        ],
        out_specs=pl.BlockSpec((rt, S), lambda r: (r, 0)),
        compiler_params=pltpu.CompilerParams(
            dimension_semantics=("parallel",),
            vmem_limit_bytes=48 * 1024 * 1024,
        ),
    )(x2, w)

    return out2.reshape(B, C, D, H, W)


# speedup vs baseline: 4.5646x; 4.2387x over previous
import functools

import jax
import jax.numpy as jnp
from jax.experimental import pallas as pl
from jax.experimental.pallas import tpu as pltpu


def _fused_kernel(x_ref, w_ref, o_ref, *, inv_s, n_c):
    xb = x_ref[0]
    m = jnp.sum(xb, axis=0, keepdims=True) * inv_s

    lane = jax.lax.broadcasted_iota(jnp.int32, m.shape, 1)
    m_prev = jnp.where(lane == 0, 0.0, jnp.roll(m, 1, axis=1))
    m_next = jnp.where(lane == n_c - 1, 0.0, jnp.roll(m, -1, axis=1))
    z = w_ref[0] * m_prev + w_ref[1] * m + w_ref[2] * m_next
    scale = 1.0 / (1.0 + jnp.exp(-z))

    t = xb * scale
    sig = 1.0 / (1.0 + jnp.exp(-t))
    o_ref[0] = (xb * sig).astype(o_ref.dtype)


def kernel(x, conv_w):
    B, C, D, H, W = x.shape
    S = D * H * W
    dtype = x.dtype

    x3 = jnp.transpose(x, (0, 2, 3, 4, 1)).reshape(B, S, C)
    w = conv_w.astype(jnp.float32)

    out3 = pl.pallas_call(
        functools.partial(_fused_kernel, inv_s=1.0 / float(S), n_c=C),
        out_shape=jax.ShapeDtypeStruct((B, S, C), dtype),
        grid=(B,),
        in_specs=[
            pl.BlockSpec((1, S, C), lambda b: (b, 0, 0)),
            pl.BlockSpec(memory_space=pltpu.SMEM),
        ],
        out_specs=pl.BlockSpec((1, S, C), lambda b: (b, 0, 0)),
        compiler_params=pltpu.CompilerParams(
            dimension_semantics=("parallel",),
            vmem_limit_bytes=48 * 1024 * 1024,
        ),
    )(x3, w)

    return jnp.transpose(out3.reshape(B, D, H, W, C), (0, 4, 1, 2, 3))


# speedup vs baseline: 5.6129x; 1.2296x over previous
import functools

import jax
import jax.numpy as jnp
from jax.experimental import pallas as pl
from jax.experimental.pallas import tpu as pltpu


def _fused_kernel(x_ref, w_ref, o_ref, *, inv_s, n_c):
    bt = x_ref.shape[0]
    lane1 = jax.lax.broadcasted_iota(jnp.int32, (1, n_c), 1)
    for i in range(bt):
        xb = x_ref[i]
        m = jnp.sum(xb, axis=0, keepdims=True) * inv_s

        m_prev = jnp.where(lane1 == 0, 0.0, jnp.roll(m, 1, axis=1))
        m_next = jnp.where(lane1 == n_c - 1, 0.0, jnp.roll(m, -1, axis=1))
        z = w_ref[0] * m_prev + w_ref[1] * m + w_ref[2] * m_next
        scale = 1.0 / (1.0 + jnp.exp(-z))

        t = xb * scale
        sig = 1.0 / (1.0 + jnp.exp(-t))
        o_ref[i] = (xb * sig).astype(o_ref.dtype)


def kernel(x, conv_w):
    B, C, D, H, W = x.shape
    S = D * H * W
    dtype = x.dtype

    x3 = jnp.transpose(x, (0, 2, 3, 4, 1)).reshape(B, S, C)
    w = conv_w.astype(jnp.float32)

    bt = 2 if B % 2 == 0 and B >= 4 else 1
    out3 = pl.pallas_call(
        functools.partial(_fused_kernel, inv_s=1.0 / float(S), n_c=C),
        out_shape=jax.ShapeDtypeStruct((B, S, C), dtype),
        grid=(B // bt,),
        in_specs=[
            pl.BlockSpec((bt, S, C), lambda b: (b, 0, 0)),
            pl.BlockSpec(memory_space=pltpu.SMEM),
        ],
        out_specs=pl.BlockSpec((bt, S, C), lambda b: (b, 0, 0)),
        compiler_params=pltpu.CompilerParams(
            dimension_semantics=("parallel",),
            vmem_limit_bytes=48 * 1024 * 1024,
        ),
    )(x3, w)

    return jnp.transpose(out3.reshape(B, D, H, W, C), (0, 4, 1, 2, 3))


# speedup vs baseline: 6.1242x; 1.0911x over previous
import functools

import jax
import jax.numpy as jnp
from jax.experimental import pallas as pl
from jax.experimental.pallas import tpu as pltpu


def _fused_kernel(x_ref, w_ref, o_ref, *, inv_s, n_c):
    bt = x_ref.shape[0]
    lane1 = jax.lax.broadcasted_iota(jnp.int32, (1, n_c), 1)
    for i in range(bt):
        xb = x_ref[i]
        m = jnp.sum(xb, axis=0, keepdims=True) * inv_s

        m_prev = jnp.where(lane1 == 0, 0.0, jnp.roll(m, 1, axis=1))
        m_next = jnp.where(lane1 == n_c - 1, 0.0, jnp.roll(m, -1, axis=1))
        z = w_ref[0] * m_prev + w_ref[1] * m + w_ref[2] * m_next
        scale = 1.0 / (1.0 + jnp.exp(-z))

        t = xb * scale
        sig = 1.0 / (1.0 + jnp.exp(-t))
        o_ref[i] = (xb * sig).astype(o_ref.dtype)


def kernel(x, conv_w):
    B, C, D, H, W = x.shape
    S = D * H * W
    dtype = x.dtype

    x3 = jnp.transpose(x, (0, 2, 3, 4, 1)).reshape(B, S, C)
    w = conv_w.astype(jnp.float32)

    bt = 4 if B % 4 == 0 and B >= 8 else 1
    out3 = pl.pallas_call(
        functools.partial(_fused_kernel, inv_s=1.0 / float(S), n_c=C),
        out_shape=jax.ShapeDtypeStruct((B, S, C), dtype),
        grid=(B // bt,),
        in_specs=[
            pl.BlockSpec((bt, S, C), lambda b: (b, 0, 0)),
            pl.BlockSpec(memory_space=pltpu.SMEM),
        ],
        out_specs=pl.BlockSpec((bt, S, C), lambda b: (b, 0, 0)),
        compiler_params=pltpu.CompilerParams(
            dimension_semantics=("parallel",),
            vmem_limit_bytes=48 * 1024 * 1024,
        ),
    )(x3, w)

    return jnp.transpose(out3.reshape(B, D, H, W, C), (0, 4, 1, 2, 3))
